# baseline (speedup 1.0000x reference)
import functools

import jax
import jax.numpy as jnp
from jax import lax
from jax.experimental import pallas as pl
from jax.experimental.pallas import tpu as pltpu
from jax.experimental.pallas import tpu_sc as plsc

_BATCH = 1024
_SEQ = 200
_N = _BATCH * _SEQ
_NC = 2
_NS = 16
_NW = _NC * _NS
_ROWS_W = _BATCH // _NW
_PER_W = _N // _NW
_CHUNK = 128
_TAIL = _SEQ - _CHUNK
_EDIM = 64
_PDIM = 128
_MDIM = 512
_MULT_A = 36313
_MULT_B = 27191
_MOD = 999999


def _hash_body(tok_ref, head_ref, tail_ref):
    t = tok_ref[...]
    prev = jnp.concatenate([t[:, :1], t[:, :-1]], axis=1)
    h = (_MULT_A * t ^ _MULT_B * prev) % _MOD
    col = lax.broadcasted_iota(jnp.int32, t.shape, 1)
    idx = jnp.where(col == 0, _MOD, h)
    head_ref[...] = idx[:, :_CHUNK]
    tail_ref[...] = jnp.concatenate(
        [idx[:, _CHUNK:],
         jnp.zeros(t.shape[:1] + (_CHUNK - _TAIL,), jnp.int32)], axis=1)


_HB = 128


@jax.jit
def _tc_hash(tok2d):
    return pl.pallas_call(
        _hash_body,
        grid=(_BATCH // _HB,),
        in_specs=[pl.BlockSpec((_HB, _SEQ), lambda i: (i, 0))],
        out_specs=[
            pl.BlockSpec((_HB, _CHUNK), lambda i: (i, 0)),
            pl.BlockSpec((_HB, _CHUNK), lambda i: (i, 0)),
        ],
        out_shape=[
            jax.ShapeDtypeStruct((_BATCH, _CHUNK), jnp.int32),
            jax.ShapeDtypeStruct((_BATCH, _CHUNK), jnp.int32),
        ],
    )(tok2d)


def _gather_body(head_hbm, tail_hbm, table_hbm, out_hbm,
                 head_v, tail_v, rows_v, rows2_v, sem):
    wid = lax.axis_index("s") * _NC + lax.axis_index("c")
    base = wid * _PER_W
    row0 = wid * _ROWS_W

    pltpu.sync_copy(head_hbm.at[pl.ds(row0, _ROWS_W)], head_v)
    pltpu.sync_copy(tail_hbm.at[pl.ds(row0, _ROWS_W)], tail_v)

    def gather_step(r, _):
        dst = base + r * _SEQ
        pltpu.async_copy(table_hbm.at[head_v.at[r]], rows_v, sem).wait()
        pltpu.sync_copy(
            rows_v, out_hbm.at[pl.ds(dst, _CHUNK), pl.ds(0, _EDIM)])
        pltpu.async_copy(table_hbm.at[tail_v.at[r]], rows2_v, sem).wait()
        pltpu.sync_copy(
            rows2_v.at[pl.ds(0, _TAIL)],
            out_hbm.at[pl.ds(dst + _CHUNK, _TAIL), pl.ds(0, _EDIM)])
        return 0

    lax.fori_loop(0, _ROWS_W, gather_step, 0)


@jax.jit
def _sc_gather(head, tail, table):
    mesh = plsc.VectorSubcoreMesh(
        core_axis_name="c", subcore_axis_name="s", num_cores=_NC,
        num_subcores=_NS)
    f = pl.kernel(
        _gather_body,
        out_type=jax.ShapeDtypeStruct((_N, _PDIM), jnp.float32),
        mesh=mesh,
        scratch_types=[
            pltpu.VMEM((_ROWS_W, _CHUNK), jnp.int32),
            pltpu.VMEM((_ROWS_W, _CHUNK), jnp.int32),
            pltpu.VMEM((_CHUNK, _EDIM), jnp.float32),
            pltpu.VMEM((_CHUNK, _EDIM), jnp.float32),
            pltpu.SemaphoreType.DMA,
        ],
        compiler_params=pltpu.CompilerParams(use_tc_tiling_on_sc=False),
    )
    return f(head, tail, table)


_RB = 1024


def _mm_body(h_ref, w_ref, o_ref):
    o_ref[...] = jnp.dot(h_ref[:, :_EDIM], w_ref[...],
                         preferred_element_type=jnp.float32)


@jax.jit
def _tc_project(h, w):
    return pl.pallas_call(
        _mm_body,
        grid=(_N // _RB,),
        in_specs=[
            pl.BlockSpec((_RB, _PDIM), lambda i: (i, 0)),
            pl.BlockSpec((_EDIM, _MDIM), lambda i: (0, 0)),
        ],
        out_specs=pl.BlockSpec((_RB, _MDIM), lambda i: (i, 0)),
        out_shape=jax.ShapeDtypeStruct((_N, _MDIM), jnp.float32),
    )(h, w)


def kernel(token_ids, embed_weight, proj_weight, scale):
    head, tail = _tc_hash(token_ids)
    gathered = _sc_gather(head, tail, embed_weight)
    w = (proj_weight * scale).T
    out = _tc_project(gathered, w)
    return out.reshape(_BATCH, _SEQ, _MDIM)


# speedup vs baseline: 2.1486x; 2.1486x over previous
import functools

import jax
import jax.numpy as jnp
from jax import lax
from jax.experimental import pallas as pl
from jax.experimental.pallas import tpu as pltpu
from jax.experimental.pallas import tpu_sc as plsc

_BATCH = 1024
_SEQ = 200
_N = _BATCH * _SEQ
_NC = 2
_NS = 16
_NW = _NC * _NS
_PER_W = _N // _NW
_CHUNK = 128
_NCHUNK = _PER_W // _CHUNK
_HVEC = _PER_W // 16
_EDIM = 64
_PDIM = 128
_MDIM = 512
_MULT_A = 36313
_MULT_B = 27191
_MOD = 999999


def _sc_body(tok_hbm, table_hbm, out_hbm, tok_v, idx_v, rows_v, sem):
    wid = lax.axis_index("s") * _NC + lax.axis_index("c")
    base = wid * _PER_W

    pltpu.sync_copy(tok_hbm.at[pl.ds(base, _PER_W)], tok_v.at[pl.ds(8, _PER_W)])

    def hash_step(k, _):
        cur = tok_v[pl.ds(8 + k * 16, 16)]
        prev = tok_v[pl.ds(7 + k * 16, 16)]
        h = (_MULT_A * cur ^ _MULT_B * prev) % _MOD
        pos = k * 16 + lax.iota(jnp.int32, 16)
        idx_v[pl.ds(k * 16, 16)] = jnp.where(pos % _SEQ == 0, _MOD, h)
        return 0

    lax.fori_loop(0, _HVEC, hash_step, 0)

    def gather_step(c, _):
        pltpu.async_copy(
            table_hbm.at[idx_v.at[pl.ds(c * _CHUNK, _CHUNK)]], rows_v,
            sem).wait()
        pltpu.sync_copy(
            rows_v,
            out_hbm.at[pl.ds(base + c * _CHUNK, _CHUNK), pl.ds(0, _EDIM)])
        return 0

    lax.fori_loop(0, _NCHUNK, gather_step, 0)


@jax.jit
def _sc_hash_gather(tok_flat, table):
    mesh = plsc.VectorSubcoreMesh(
        core_axis_name="c", subcore_axis_name="s", num_cores=_NC,
        num_subcores=_NS)
    f = pl.kernel(
        _sc_body,
        out_type=jax.ShapeDtypeStruct((_N, _PDIM), jnp.float32),
        mesh=mesh,
        scratch_types=[
            pltpu.VMEM((_PER_W + 8,), jnp.int32),
            pltpu.VMEM((_PER_W,), jnp.int32),
            pltpu.VMEM((_CHUNK, _EDIM), jnp.float32),
            pltpu.SemaphoreType.DMA,
        ],
        compiler_params=pltpu.CompilerParams(use_tc_tiling_on_sc=False),
    )
    return f(tok_flat, table)


_RB = 1024


def _mm_body(h_ref, w_ref, o_ref):
    o_ref[...] = jnp.dot(h_ref[:, :_EDIM], w_ref[...],
                         preferred_element_type=jnp.float32)


@jax.jit
def _tc_project(h, w):
    return pl.pallas_call(
        _mm_body,
        grid=(_N // _RB,),
        in_specs=[
            pl.BlockSpec((_RB, _PDIM), lambda i: (i, 0)),
            pl.BlockSpec((_EDIM, _MDIM), lambda i: (0, 0)),
        ],
        out_specs=pl.BlockSpec((_RB, _MDIM), lambda i: (i, 0)),
        out_shape=jax.ShapeDtypeStruct((_N, _MDIM), jnp.float32),
    )(h, w)


def kernel(token_ids, embed_weight, proj_weight, scale):
    ii = jnp.arange(_N, dtype=jnp.int32)
    tok_flat = token_ids[ii // _SEQ, ii % _SEQ]
    gathered = _sc_hash_gather(tok_flat, embed_weight)
    w = (proj_weight * scale).T
    out = _tc_project(gathered, w)
    return out.reshape(_BATCH, _SEQ, _MDIM)
